# Initial kernel scaffold; baseline (speedup 1.0000x reference)
import jax
import jax.numpy as jnp
from jax.experimental import pallas as pl


def kernel(x, edge_index, W_gcn, b_gcn, W_gat, att_src, att_dst, b_gat, W_fc, b_fc):
    raise NotImplementedError("write your pallas kernel here")


# speedup vs baseline: 1.1368x; 1.1368x over previous
import functools

import jax
import jax.numpy as jnp
from jax import lax
from jax.experimental import pallas as pl
from jax.experimental.pallas import tpu as pltpu, tpu_sc as plsc

N = 10000
E = 160000
IN_C, HID_C, OUT_C = 256, 512, 256
NC, NS, L = 2, 16, 16
NW = NC * NS
EPW = E // NW


def _deg_body(dst_hbm, out_hbm, dst_v, hist_v, sem):
    c = lax.axis_index("c")
    s = lax.axis_index("s")
    wid = c * NS + s

    zeros = jnp.zeros((L,), jnp.int32)

    def zloop(i, _):
        hist_v[pl.ds(i * L, L)] = zeros
        return 0

    lax.fori_loop(0, N // L, zloop, 0)

    pltpu.sync_copy(dst_hbm.at[pl.ds(wid * EPW, EPW)], dst_v)

    ones = jnp.ones((L,), jnp.int32)

    def body(i, _):
        idx = dst_v[pl.ds(i * L, L)]
        plsc.addupdate_scatter(hist_v, [idx], ones)
        return 0

    lax.fori_loop(0, EPW // L, body, 0)

    pltpu.sync_copy(hist_v, out_hbm.at[wid])


_deg_kernel = functools.partial(
    pl.kernel,
    out_type=jax.ShapeDtypeStruct((NW, N), jnp.int32),
    mesh=plsc.VectorSubcoreMesh(core_axis_name="c", subcore_axis_name="s", num_cores=NC, num_subcores=NS),
    scratch_types=[
        pltpu.VMEM((EPW,), jnp.int32),
        pltpu.VMEM((N,), jnp.int32),
        pltpu.SemaphoreType.DMA,
    ],
    compiler_params=pltpu.CompilerParams(needs_layout_passes=False),
)(_deg_body)


NCH = 2 * NW
CW = 157
RWS = 160
DUMP = CW + 1
BLK = 2000
NV = BLK // L
G = 16
ACC = RWS * HID_C


def _make_agg(gat: bool):
    def body(rows_hbm, src_hbm, dst_hbm, a_s_hbm, a_d_hbm,
             out_hbm, den_hbm, sstage, dstage, psrc, pdst, pw,
             as_tab, ad_tab, rowbuf, acc, den, sem):
        c = lax.axis_index("c")
        s = lax.axis_index("s")
        w_id = c * NS + s

        if gat:
            pltpu.sync_copy(a_s_hbm, as_tab)
            pltpu.sync_copy(a_d_hbm, ad_tab)

        zv = jnp.zeros((L,), jnp.float32)
        zi16 = jnp.zeros((L,), jnp.int32)
        iota16 = lax.iota(jnp.int32, L)

        for chunk_i in range(NCH // NW):
            chunk = chunk_i * NW + w_id
            lo = chunk * CW

            def zbody(i, _):
                for u in range(8):
                    acc[pl.ds((i * 8 + u) * L, L)] = zv
                return 0
            lax.fori_loop(0, ACC // (8 * L), zbody, 0)
            if gat:
                for u in range(RWS // L):
                    den[pl.ds(u * L, L)] = zv

            def blk_body(blk, _):
                ebase = blk * BLK
                pltpu.sync_copy(src_hbm.at[pl.ds(ebase, BLK)], sstage)
                pltpu.sync_copy(dst_hbm.at[pl.ds(ebase, BLK)], dstage)

                def scan_body(v, cnt):
                    srcv = sstage[pl.ds(v * L, L)]
                    dstv = dstage[pl.ds(v * L, L)]
                    dl = dstv - lo
                    mask = (dl >= 0) & (dl < CW)
                    plsc.store_compressed(psrc.at[pl.ds(cnt, L)], srcv, mask=mask)
                    plsc.store_compressed(pdst.at[pl.ds(cnt, L)], dl, mask=mask)
                    if gat:
                        asv = plsc.load_gather(as_tab, [srcv])
                        adv = plsc.load_gather(ad_tab, [dstv])
                        z = asv + adv
                        wv = jnp.exp(jnp.where(z > 0, z, 0.2 * z))
                        plsc.store_compressed(pw.at[pl.ds(cnt, L)], wv, mask=mask)
                    return cnt + jnp.sum(mask.astype(jnp.int32))

                cnt = lax.fori_loop(0, NV, scan_body, jnp.int32(0))

                psrc[pl.ds(cnt, L)] = jnp.zeros((L,), jnp.int32)
                pdst[pl.ds(cnt, L)] = jnp.full((L,), DUMP, jnp.int32)
                if gat:
                    pw[pl.ds(cnt, L)] = zv

                nb = (cnt + (G - 1)) // G

                def flush_body(b, _):
                    off = b * G
                    pltpu.async_copy(rows_hbm.at[psrc.at[pl.ds(off, G)]],
                                     rowbuf, sem).wait()
                    dl16 = pdst[pl.ds(off, L)]
                    abase = dl16 * HID_C
                    if gat:
                        w16 = pw[pl.ds(off, L)]
                        plsc.addupdate_scatter(den, [dl16], w16)

                    def col_body(ob, _):
                        cb = ob * 8
                        for k in range(8):
                            cc = cb + k
                            val = plsc.load_gather(rowbuf, [iota16, zi16 + cc])
                            if gat:
                                val = val * w16
                            plsc.addupdate_scatter(acc, [abase + cc], val)
                        return 0
                    lax.fori_loop(0, HID_C // 8, col_body, 0)
                    return 0

                lax.fori_loop(0, nb, flush_body, 0)
                return 0

            lax.fori_loop(0, E // BLK, blk_body, 0)

            pltpu.sync_copy(acc, out_hbm.at[pl.ds(chunk * ACC, ACC)])
            if gat:
                pltpu.sync_copy(den, den_hbm.at[pl.ds(chunk * RWS, RWS)])

    scratch = [
        pltpu.VMEM((BLK,), jnp.int32),
        pltpu.VMEM((BLK,), jnp.int32),
        pltpu.VMEM((BLK + 2 * L,), jnp.int32),
        pltpu.VMEM((BLK + 2 * L,), jnp.int32),
        pltpu.VMEM((BLK + 2 * L,), jnp.float32) if gat else None,
        pltpu.VMEM((N,), jnp.float32) if gat else None,
        pltpu.VMEM((N,), jnp.float32) if gat else None,
        pltpu.VMEM((G, HID_C), jnp.float32),
        pltpu.VMEM((ACC,), jnp.float32),
        pltpu.VMEM((RWS,), jnp.float32) if gat else None,
        pltpu.SemaphoreType.DMA,
    ]

    out_type = [jax.ShapeDtypeStruct((NCH * ACC,), jnp.float32)]
    if gat:
        out_type.append(jax.ShapeDtypeStruct((NCH * RWS,), jnp.float32))

    def wrapped(rows, src, dst, a_s, a_d):
        if not gat:
            a_s = jnp.zeros((8,), jnp.float32)
            a_d = jnp.zeros((8,), jnp.float32)

        def body2(rows_hbm, src_hbm, dst_hbm, a_s_hbm, a_d_hbm, *rest):
            rest = list(rest)
            out_hbm = rest.pop(0)
            den_hbm = rest.pop(0) if gat else None
            it = iter(rest)
            full = []
            for sp in scratch[:-1]:
                full.append(None if sp is None else next(it))
            sem = next(it)
            body(rows_hbm, src_hbm, dst_hbm, a_s_hbm, a_d_hbm,
                 out_hbm, den_hbm, *full, sem)

        return pl.kernel(
            body2,
            out_type=out_type if gat else out_type[0],
            mesh=plsc.VectorSubcoreMesh(core_axis_name="c", subcore_axis_name="s", num_cores=NC, num_subcores=NS),
            scratch_types=[sp for sp in scratch if sp is not None],
            compiler_params=pltpu.CompilerParams(needs_layout_passes=False),
        )(rows, src, dst, a_s, a_d)

    return wrapped


_gcn_agg = _make_agg(gat=False)
_gat_agg = _make_agg(gat=True)


MB = 1000


def _tc1_body(parts_ref, x_ref, w_ref, y_ref, dis_ref):
    deg = jnp.sum(parts_ref[...], axis=1, keepdims=True).astype(jnp.float32) + 1.0
    dis = lax.rsqrt(deg)
    xw = jnp.dot(x_ref[...], w_ref[...], preferred_element_type=jnp.float32)
    y_ref[...] = xw * dis
    dis_ref[...] = dis


def _tc1(deg_parts, x, W_gcn):
    return pl.pallas_call(
        _tc1_body,
        grid=(N // MB,),
        in_specs=[
            pl.BlockSpec((MB, NW), lambda i: (i, 0)),
            pl.BlockSpec((MB, IN_C), lambda i: (i, 0)),
            pl.BlockSpec((IN_C, HID_C), lambda i: (0, 0)),
        ],
        out_specs=[
            pl.BlockSpec((MB, HID_C), lambda i: (i, 0)),
            pl.BlockSpec((MB, 1), lambda i: (i, 0)),
        ],
        out_shape=[
            jax.ShapeDtypeStruct((N, HID_C), jnp.float32),
            jax.ShapeDtypeStruct((N, 1), jnp.float32),
        ],
    )(deg_parts, x, W_gcn)


def _tc2_body(acc_ref, y_ref, dis_ref, bg_ref, w_ref, as_ref, ad_ref,
              g_ref, asn_ref, adn_ref):
    h = jnp.maximum(dis_ref[...] * (acc_ref[...] + y_ref[...]) + bg_ref[...], 0.0)
    g = jnp.dot(h, w_ref[...], preferred_element_type=jnp.float32)
    g_ref[...] = g
    asn_ref[...] = jnp.sum(g * as_ref[...], axis=-1, keepdims=True)
    adn_ref[...] = jnp.sum(g * ad_ref[...], axis=-1, keepdims=True)


def _tc2(acc, y, dis, b_gcn, W_gat, att_src, att_dst):
    return pl.pallas_call(
        _tc2_body,
        grid=(N // MB,),
        in_specs=[
            pl.BlockSpec((MB, HID_C), lambda i: (i, 0)),
            pl.BlockSpec((MB, HID_C), lambda i: (i, 0)),
            pl.BlockSpec((MB, 1), lambda i: (i, 0)),
            pl.BlockSpec((1, HID_C), lambda i: (0, 0)),
            pl.BlockSpec((HID_C, HID_C), lambda i: (0, 0)),
            pl.BlockSpec((1, HID_C), lambda i: (0, 0)),
            pl.BlockSpec((1, HID_C), lambda i: (0, 0)),
        ],
        out_specs=[
            pl.BlockSpec((MB, HID_C), lambda i: (i, 0)),
            pl.BlockSpec((MB, 1), lambda i: (i, 0)),
            pl.BlockSpec((MB, 1), lambda i: (i, 0)),
        ],
        out_shape=[
            jax.ShapeDtypeStruct((N, HID_C), jnp.float32),
            jax.ShapeDtypeStruct((N, 1), jnp.float32),
            jax.ShapeDtypeStruct((N, 1), jnp.float32),
        ],
    )(acc, y, dis, b_gcn[None, :], W_gat, att_src[None, :], att_dst[None, :])


def _tc3_body(vacc_ref, den_ref, g_ref, as_ref, ad_ref, bg_ref, w_ref, bf_ref,
              out_ref):
    z = as_ref[...] + ad_ref[...]
    ws = jnp.exp(jnp.where(z > 0, z, 0.2 * z))
    outp = (vacc_ref[...] + ws * g_ref[...]) / (den_ref[...] + ws) + bg_ref[...]
    outp = jnp.maximum(outp, 0.0)
    out_ref[...] = (jnp.dot(outp, w_ref[...], preferred_element_type=jnp.float32)
                    + bf_ref[...])


def _tc3(vacc, den, g, a_s, a_d, b_gat, W_fc, b_fc):
    return pl.pallas_call(
        _tc3_body,
        grid=(N // MB,),
        in_specs=[
            pl.BlockSpec((MB, HID_C), lambda i: (i, 0)),
            pl.BlockSpec((MB, 1), lambda i: (i, 0)),
            pl.BlockSpec((MB, HID_C), lambda i: (i, 0)),
            pl.BlockSpec((MB, 1), lambda i: (i, 0)),
            pl.BlockSpec((MB, 1), lambda i: (i, 0)),
            pl.BlockSpec((1, HID_C), lambda i: (0, 0)),
            pl.BlockSpec((HID_C, OUT_C), lambda i: (0, 0)),
            pl.BlockSpec((1, OUT_C), lambda i: (0, 0)),
        ],
        out_specs=pl.BlockSpec((MB, OUT_C), lambda i: (i, 0)),
        out_shape=jax.ShapeDtypeStruct((N, OUT_C), jnp.float32),
    )(vacc, den, g, a_s, a_d, b_gat[None, :], W_fc, b_fc[None, :])


def kernel(x, edge_index, W_gcn, b_gcn, W_gat, att_src, att_dst, b_gat, W_fc, b_fc):
    src = edge_index[0].astype(jnp.int32)
    dst = edge_index[1].astype(jnp.int32)

    deg_parts = _deg_kernel(dst)

    y, dis = _tc1(deg_parts.T, x, W_gcn)
    agg = _gcn_agg(y, src, dst, None, None)
    agg = agg.reshape(NCH, RWS, HID_C)[:, :CW, :].reshape(NCH * CW, HID_C)[:N]

    g, a_s2, a_d2 = _tc2(agg, y, dis, b_gcn, W_gat, att_src, att_dst)
    vacc, den = _gat_agg(g, src, dst, a_s2[:, 0], a_d2[:, 0])
    vacc = vacc.reshape(NCH, RWS, HID_C)[:, :CW, :].reshape(NCH * CW, HID_C)[:N]
    den = den.reshape(NCH, RWS)[:, :CW].reshape(NCH * CW)[:N, None]

    return _tc3(vacc, den, g, a_s2, a_d2, b_gat, W_fc, b_fc)
